# Initial kernel scaffold; baseline (speedup 1.0000x reference)
import jax
import jax.numpy as jnp
from jax.experimental import pallas as pl


def kernel(x, r_W, W_qkv, W_o, proj_W, proj_b):
    raise NotImplementedError("write your pallas kernel here")


# speedup vs baseline: 3.3773x; 3.3773x over previous
import jax
import jax.numpy as jnp
import numpy as np
from jax.experimental import pallas as pl

_DIM = 96
_HEADS = 6
_HD = 16
_WS = 16
_L = _WS * _WS
_K1 = _L // 2 - 1
_SCALE = 1.0 / np.sqrt(_HD)
_NEG = -1e30


def _wmosa_body(xw_ref, rW_ref, wqkv_ref, wo_ref, pW_ref, pb_ref, out_ref):
    xw = xw_ref[0]
    logits = jax.nn.sigmoid(
        jnp.dot(xw, rW_ref[...], preferred_element_type=jnp.float32))
    qkv = jnp.dot(xw, wqkv_ref[...], preferred_element_type=jnp.float32)

    ii = jax.lax.broadcasted_iota(jnp.int32, (_L, _L), 0)
    jj = jax.lax.broadcasted_iota(jnp.int32, (_L, _L), 1)
    jrow = jax.lax.broadcasted_iota(jnp.int32, (1, _L), 1)

    outs = []
    for h in range(_HEADS):
        v_col = logits[:, h:h + 1]
        v_row = jnp.transpose(v_col)
        beats = ((v_col > v_row) | ((v_col == v_row) & (ii < jj))) & (ii >= 1)
        rank = jnp.sum(beats.astype(jnp.float32), axis=0, keepdims=True)
        mask_row = (jrow == 0) | ((jrow >= 1) & (rank < _K1))
        kbias = jnp.where(mask_row, 0.0, _NEG)
        mask_col = jnp.transpose(mask_row)

        q = qkv[:, h * 48:h * 48 + 16]
        k = qkv[:, h * 48 + 16:h * 48 + 32]
        val = qkv[:, h * 48 + 32:h * 48 + 48]
        scores = jax.lax.dot_general(
            q, k, (((1,), (1,)), ((), ())),
            preferred_element_type=jnp.float32) * _SCALE + kbias
        m = jnp.max(scores, axis=1, keepdims=True)
        p = jnp.exp(scores - m)
        attn = p / jnp.sum(p, axis=1, keepdims=True)
        o = jnp.dot(attn, val, preferred_element_type=jnp.float32)
        gate = jnp.where(mask_col, v_col, 0.0)
        outs.append(o * gate)

    ocat = jnp.concatenate(outs, axis=1)
    pres = jnp.dot(ocat, wo_ref[...], preferred_element_type=jnp.float32)
    y = jnp.dot(pres, pW_ref[...], preferred_element_type=jnp.float32) + pb_ref[...]
    out_ref[0] = y


def kernel(x, r_W, W_qkv, W_o, proj_W, proj_b):
    B, H, W, C = x.shape
    nh, nw = H // _WS, W // _WS
    xw = x.reshape(B, nh, _WS, nw, _WS, C).transpose(0, 1, 3, 2, 4, 5)
    xw = xw.reshape(B * nh * nw, _L, C)
    Bn = xw.shape[0]

    wqkv = jnp.transpose(W_qkv, (1, 0, 2)).reshape(C, _HEADS * 3 * _HD)
    wo = W_o.reshape(_HEADS * _HD, C)
    pb = proj_b.reshape(1, C)

    y = pl.pallas_call(
        _wmosa_body,
        grid=(Bn,),
        in_specs=[
            pl.BlockSpec((1, _L, C), lambda i: (i, 0, 0)),
            pl.BlockSpec((C, _HEADS), lambda i: (0, 0)),
            pl.BlockSpec((C, _HEADS * 3 * _HD), lambda i: (0, 0)),
            pl.BlockSpec((_HEADS * _HD, C), lambda i: (0, 0)),
            pl.BlockSpec((C, C), lambda i: (0, 0)),
            pl.BlockSpec((1, C), lambda i: (0, 0)),
        ],
        out_specs=pl.BlockSpec((1, _L, C), lambda i: (i, 0, 0)),
        out_shape=jax.ShapeDtypeStruct((Bn, _L, C), x.dtype),
    )(xw, r_W, wqkv, wo, proj_W, pb)

    x_out = y.reshape(B, nh, nw, _WS, _WS, C).transpose(0, 1, 3, 2, 4, 5)
    return x_out.reshape(B, H, W, C)


# speedup vs baseline: 4.3448x; 1.2865x over previous
import jax
import jax.numpy as jnp
import numpy as np
from jax.experimental import pallas as pl

_DIM = 96
_HEADS = 6
_HD = 16
_WS = 16
_L = _WS * _WS
_K1 = _L // 2 - 1
_SCALE = 1.0 / np.sqrt(_HD)
_NEG = -1e30


_WPB = 4


def _wmosa_body(xw_ref, rW_ref, wqkv_ref, wo_ref, pW_ref, pb_ref, out_ref):
    ii = jax.lax.broadcasted_iota(jnp.int32, (_L, _L), 0)
    jj = jax.lax.broadcasted_iota(jnp.int32, (_L, _L), 1)
    jrow = jax.lax.broadcasted_iota(jnp.int32, (1, _L), 1)

    for w in range(_WPB):
        xw = xw_ref[w]
        logits = jax.nn.sigmoid(
            jnp.dot(xw, rW_ref[...], preferred_element_type=jnp.float32))
        logitsT = jnp.transpose(logits)
        qkv = jnp.dot(xw, wqkv_ref[...], preferred_element_type=jnp.float32)

        outs = []
        for h in range(_HEADS):
            v_col = logits[:, h:h + 1]
            v_row = logitsT[h:h + 1, :]
            beats = ((v_col > v_row) | ((v_col == v_row) & (ii < jj))) & (ii >= 1)
            rank = jnp.sum(beats.astype(jnp.float32), axis=0, keepdims=True)
            mask_row = (jrow == 0) | ((jrow >= 1) & (rank < _K1))
            kbias = jnp.where(mask_row, 0.0, _NEG)
            mask_col = jnp.transpose(mask_row)

            q = qkv[:, h * 48:h * 48 + 16]
            k = qkv[:, h * 48 + 16:h * 48 + 32]
            val = qkv[:, h * 48 + 32:h * 48 + 48]
            scores = jax.lax.dot_general(
                q, k, (((1,), (1,)), ((), ())),
                preferred_element_type=jnp.float32) * _SCALE + kbias
            m = jnp.max(scores, axis=1, keepdims=True)
            p = jnp.exp(scores - m)
            attn = p / jnp.sum(p, axis=1, keepdims=True)
            o = jnp.dot(attn, val, preferred_element_type=jnp.float32)
            gate = jnp.where(mask_col, v_col, 0.0)
            outs.append(o * gate)

        ocat = jnp.concatenate(outs, axis=1)
        pres = jnp.dot(ocat, wo_ref[...], preferred_element_type=jnp.float32)
        y = jnp.dot(pres, pW_ref[...], preferred_element_type=jnp.float32) + pb_ref[...]
        out_ref[w] = y


def kernel(x, r_W, W_qkv, W_o, proj_W, proj_b):
    B, H, W, C = x.shape
    nh, nw = H // _WS, W // _WS
    xw = x.reshape(B, nh, _WS, nw, _WS, C).transpose(0, 1, 3, 2, 4, 5)
    xw = xw.reshape(B * nh * nw, _L, C)
    Bn = xw.shape[0]

    wqkv = jnp.transpose(W_qkv, (1, 0, 2)).reshape(C, _HEADS * 3 * _HD)
    wo = W_o.reshape(_HEADS * _HD, C)
    pb = proj_b.reshape(1, C)

    y = pl.pallas_call(
        _wmosa_body,
        grid=(Bn // _WPB,),
        in_specs=[
            pl.BlockSpec((_WPB, _L, C), lambda i: (i, 0, 0)),
            pl.BlockSpec((C, _HEADS), lambda i: (0, 0)),
            pl.BlockSpec((C, _HEADS * 3 * _HD), lambda i: (0, 0)),
            pl.BlockSpec((_HEADS * _HD, C), lambda i: (0, 0)),
            pl.BlockSpec((C, C), lambda i: (0, 0)),
            pl.BlockSpec((1, C), lambda i: (0, 0)),
        ],
        out_specs=pl.BlockSpec((_WPB, _L, C), lambda i: (i, 0, 0)),
        out_shape=jax.ShapeDtypeStruct((Bn, _L, C), x.dtype),
    )(xw, r_W, wqkv, wo, proj_W, pb)

    x_out = y.reshape(B, nh, nw, _WS, _WS, C).transpose(0, 1, 3, 2, 4, 5)
    return x_out.reshape(B, H, W, C)
